# baseline (speedup 1.0000x reference)
import functools

import jax
import jax.numpy as jnp
from jax.experimental import pallas as pl
from jax.experimental.pallas import tpu as pltpu

N = 10000
E = 64000
H = 40


def _mm_body(x_ref, w_ref, o_ref, acc_ref, *, nk):
    k = pl.program_id(2)

    @pl.when(k == 0)
    def _():
        acc_ref[...] = jnp.zeros_like(acc_ref)

    acc_ref[...] += jnp.dot(x_ref[...], w_ref[...],
                            preferred_element_type=jnp.float32)

    @pl.when(k == nk - 1)
    def _():
        o_ref[...] = acc_ref[...]


def _pick(n, cap, mult):
    best = mult
    d = mult
    while d <= cap:
        if n % d == 0:
            best = d
        d += mult
    return best


def _matmul(x, w):
    M, K = x.shape
    _, Nn = w.shape
    bm = _pick(M, 512, 8)
    bn = _pick(Nn, 512, 128) if Nn >= 128 else Nn
    bk = K if K <= 1280 else _pick(K, 1280, 128)
    nk = K // bk
    return pl.pallas_call(
        functools.partial(_mm_body, nk=nk),
        grid=(M // bm, Nn // bn, nk),
        in_specs=[pl.BlockSpec((bm, bk), lambda i, j, k: (i, k)),
                  pl.BlockSpec((bk, bn), lambda i, j, k: (k, j))],
        out_specs=pl.BlockSpec((bm, bn), lambda i, j, k: (i, j)),
        out_shape=jax.ShapeDtypeStruct((M, Nn), jnp.float32),
        scratch_shapes=[pltpu.VMEM((bm, bn), jnp.float32)],
        compiler_params=pltpu.CompilerParams(
            dimension_semantics=("parallel", "parallel", "arbitrary")),
    )(x, w)


def _gatv2(p, x, src, dst, edge_attr, C):
    xl = _matmul(x, p['Wl']).reshape(N, H, C)
    xr = _matmul(x, p['Wr']).reshape(N, H, C)
    e = _matmul(edge_attr, p['We']).reshape(E, H, C)
    m = xl[src] + xr[dst] + e
    m = jnp.where(m > 0, m, 0.0)
    alpha = jnp.sum(m * p['att'][None], axis=-1)
    amax = jax.ops.segment_max(alpha, dst, num_segments=N)
    amax = jnp.where(jnp.isfinite(amax), amax, 0.0)
    a = jnp.exp(alpha - amax[dst])
    den = jax.ops.segment_sum(a, dst, num_segments=N)
    a = a / (den[dst] + 1e-16)
    out = jax.ops.segment_sum(xl[src] * a[:, :, None], dst, num_segments=N)
    return out.reshape(N, H * C) + _matmul(x, p['Wres']) + p['b']


def _bn(h, p):
    mu = jnp.mean(h, axis=0)
    var = jnp.var(h, axis=0)
    return p['g'] * (h - mu) / jnp.sqrt(var + 1e-5) + p['b']


def kernel(x, edge_index, edge_attr, batch, params):
    src, dst = edge_index[0], edge_index[1]
    h = jax.nn.selu(_gatv2(params['gat1'], x, src, dst, edge_attr, 16))
    h = jax.nn.selu(_gatv2(params['gat2'], h, src, dst, edge_attr, 32))
    h = jax.nn.selu(_gatv2(params['gat3'], h, src, dst, edge_attr, 64))
    h = _bn(_matmul(h, params['lin1']['W']) + params['lin1']['b'], params['bn1'])
    h = jax.nn.selu(h)
    h = _bn(_matmul(h, params['lin2']['W']) + params['lin2']['b'], params['bn2'])
    h = jax.nn.selu(h)
    RF = _matmul(h, params['lin3']['W']) + params['lin3']['b']
    BB = _matmul(h, params['lin4']['W']) + params['lin4']['b']
    P = _matmul(h, params['lin5']['W']) + params['lin5']['b']
    return (RF, BB, P)


# speedup vs baseline: 1.2660x; 1.2660x over previous
import functools

import jax
import jax.numpy as jnp
from jax import lax
from jax.experimental import pallas as pl
from jax.experimental.pallas import tpu as pltpu
from jax.experimental.pallas import tpu_sc as plsc

N = 10000
E = 64000
H = 40
HP = 48
NC = 2
NS = 16
NW = NC * NS
CE = E // NW
NN = 320
NP = NW * NN
EB = 8
EP = E + EB

SELU_L = 1.0507009873554805
SELU_A = 1.6732632423543772

_MESH = dict(core_axis_name="c", subcore_axis_name="s")


def _mm_body(x_ref, w_ref, o_ref, acc_ref, *, nk):
    k = pl.program_id(2)

    @pl.when(k == 0)
    def _():
        acc_ref[...] = jnp.zeros_like(acc_ref)

    acc_ref[...] += jnp.dot(x_ref[...], w_ref[...],
                            preferred_element_type=jnp.float32)

    @pl.when(k == nk - 1)
    def _():
        o_ref[...] = acc_ref[...]


def _pick(n, cap, mult):
    best = mult
    d = mult
    while d <= cap:
        if n % d == 0:
            best = d
        d += mult
    return best


def _matmul(x, w):
    M, K = x.shape
    _, Nn = w.shape
    bm = _pick(M, 512, 8)
    bn = _pick(Nn, 512, 128) if Nn >= 128 else Nn
    bk = K if K <= 1280 else _pick(K, 1280, 128)
    nk = K // bk
    return pl.pallas_call(
        functools.partial(_mm_body, nk=nk),
        grid=(M // bm, Nn // bn, nk),
        in_specs=[pl.BlockSpec((bm, bk), lambda i, j, k: (i, k)),
                  pl.BlockSpec((bk, bn), lambda i, j, k: (k, j))],
        out_specs=pl.BlockSpec((bm, bn), lambda i, j, k: (i, j)),
        out_shape=jax.ShapeDtypeStruct((M, Nn), jnp.float32),
        scratch_shapes=[pltpu.VMEM((bm, bn), jnp.float32)],
        compiler_params=pltpu.CompilerParams(
            dimension_semantics=("parallel", "parallel", "arbitrary")),
    )(x, w)


def _wid():
    return lax.axis_index("s") * NC + lax.axis_index("c")


def _make_k3(C):
    HC = H * C
    mesh = plsc.VectorSubcoreMesh(**_MESH)

    @functools.partial(
        pl.kernel, mesh=mesh,
        out_type=jax.ShapeDtypeStruct((EP, HP), jnp.float32),
        scratch_types=[
            pltpu.VMEM((CE,), jnp.int32),
            pltpu.VMEM((CE,), jnp.int32),
            pltpu.VMEM((CE,), jnp.int32),
            pltpu.VMEM((HP * C,), jnp.float32),
            pltpu.VMEM((EB, HC), jnp.float32),
            pltpu.VMEM((EB, HC), jnp.float32),
            pltpu.VMEM((EB, HC), jnp.float32),
            pltpu.VMEM((EB, HP), jnp.float32),
            pltpu.SemaphoreType.DMA,
        ],
        compiler_params=pltpu.CompilerParams(needs_layout_passes=False),
    )
    def k3(xl, xr, ew, ssrc, sdst, perm, att, alpha,
           srcv, dstv, permv, attv, glb, grb, ewb, astage, sem):
        wid = _wid()
        ebase = wid * CE
        pltpu.sync_copy(ssrc.at[pl.ds(ebase, CE)], srcv)
        pltpu.sync_copy(sdst.at[pl.ds(ebase, CE)], dstv)
        pltpu.sync_copy(perm.at[pl.ds(ebase, CE)], permv)
        pltpu.sync_copy(att, attv)
        iot = lax.iota(jnp.int32, 16)

        def bbody(b, carry):
            off = b * EB
            pltpu.async_copy(xl.at[srcv.at[pl.ds(off, EB)]], glb, sem).wait()
            pltpu.async_copy(xr.at[dstv.at[pl.ds(off, EB)]], grb, sem).wait()
            pltpu.async_copy(ew.at[permv.at[pl.ds(off, EB)]], ewb, sem).wait()

            def cbody(c, accs):
                out = list(accs)
                for hb in range(3):
                    hidx = (hb * 16 + iot) * C + c
                    av = plsc.load_gather(attv, [hidx])
                    gidx = jnp.minimum(hidx, HC - 1)
                    for e in range(EB):
                        es = jnp.full((16,), e, jnp.int32)
                        m = (plsc.load_gather(glb, [es, gidx])
                             + plsc.load_gather(grb, [es, gidx])
                             + plsc.load_gather(ewb, [es, gidx]))
                        m = jnp.maximum(m, 0.0)
                        out[e * 3 + hb] = out[e * 3 + hb] + m * av
                return tuple(out)

            accs = lax.fori_loop(
                0, C, cbody,
                tuple(jnp.zeros((16,), jnp.float32) for _ in range(EB * 3)))
            for e in range(EB):
                for hb in range(3):
                    astage[e, pl.ds(hb * 16, 16)] = accs[e * 3 + hb]
            pltpu.sync_copy(astage, alpha.at[pl.ds(ebase + off, EB), :])
            return carry

        lax.fori_loop(0, CE // EB, bbody, 0)

    return k3


def _make_k4():
    mesh = plsc.VectorSubcoreMesh(**_MESH)
    neg_inf = float('-inf')

    @functools.partial(
        pl.kernel, mesh=mesh,
        out_type=(jax.ShapeDtypeStruct((NP, HP), jnp.float32),
                  jax.ShapeDtypeStruct((NP, HP), jnp.float32)),
        scratch_types=[
            pltpu.VMEM((NN + 16,), jnp.int32),
            pltpu.VMEM((EB, HP), jnp.float32),
            pltpu.VMEM((NN, HP), jnp.float32),
            pltpu.VMEM((NN, HP), jnp.float32),
        ],
        compiler_params=pltpu.CompilerParams(needs_layout_passes=False),
    )
    def k4(alpha, start, amax, den, startv, ab, mstage, dstage):
        wid = _wid()
        nbase = wid * NN
        pltpu.sync_copy(start.at[pl.ds(nbase, NN + 8)],
                        startv.at[pl.ds(0, NN + 8)])

        def nbody(i, _):
            sv = startv[pl.ds(i, 16)]
            s = sv[0]
            e1 = sv[1]
            b0 = s // EB
            b1 = (e1 + EB - 1) // EB

            def bb(b, carry):
                ms = list(carry[:3])
                ds = list(carry[3:])
                off = pl.multiple_of(b * EB, EB)
                pltpu.sync_copy(alpha.at[pl.ds(off, EB), :], ab)
                for e in range(EB):
                    valid = ((off + e) >= s) & ((off + e) < e1)
                    for k in range(3):
                        r = ab[e, pl.ds(k * 16, 16)]
                        rv = jnp.where(valid, r, neg_inf)
                        nm = jnp.maximum(ms[k], rv)
                        dn = ds[k] * jnp.exp(ms[k] - nm) + jnp.exp(rv - nm)
                        ds[k] = jnp.where(nm == neg_inf, ds[k], dn)
                        ms[k] = nm
                return tuple(ms) + tuple(ds)

            init = (tuple(jnp.full((16,), neg_inf, jnp.float32)
                          for _ in range(3))
                    + tuple(jnp.zeros((16,), jnp.float32) for _ in range(3)))
            res = lax.fori_loop(b0, b1, bb, init)
            for k in range(3):
                m = res[k]
                mm = jnp.where(m == neg_inf, 0.0, m)
                mstage[i, pl.ds(k * 16, 16)] = mm
                dstage[i, pl.ds(k * 16, 16)] = res[3 + k]
            return 0

        lax.fori_loop(0, NN, nbody, 0)
        pltpu.sync_copy(mstage, amax.at[pl.ds(nbase, NN), :])
        pltpu.sync_copy(dstage, den.at[pl.ds(nbase, NN), :])

    return k4


def _make_k5(C):
    HC = H * C
    mesh = plsc.VectorSubcoreMesh(**_MESH)

    @functools.partial(
        pl.kernel, mesh=mesh,
        out_type=jax.ShapeDtypeStruct((N, HC), jnp.float32),
        scratch_types=[
            pltpu.VMEM((NN + 16,), jnp.int32),
            pltpu.VMEM((NN, HP), jnp.float32),
            pltpu.VMEM((NN, HP), jnp.float32),
            pltpu.VMEM((EB,), jnp.int32),
            pltpu.VMEM((EB, HC), jnp.float32),
            pltpu.VMEM((EB, HP), jnp.float32),
            pltpu.VMEM((HC,), jnp.float32),
            pltpu.VMEM((HC,), jnp.float32),
            pltpu.VMEM((HC,), jnp.float32),
            pltpu.VMEM((HC,), jnp.float32),
            pltpu.SemaphoreType.DMA,
        ],
        compiler_params=pltpu.CompilerParams(needs_layout_passes=False),
    )
    def k5(alpha, start, amaxh, denh, ssrcp, xl, xres, bias, out,
           startv, amaxv, denv, idxb, glb, ab, acc, xrow, bv, orow,
           sem):
        wid = _wid()
        nbase = wid * NN
        pltpu.sync_copy(start.at[pl.ds(nbase, NN + 8)],
                        startv.at[pl.ds(0, NN + 8)])
        pltpu.sync_copy(amaxh.at[pl.ds(nbase, NN), :], amaxv)
        pltpu.sync_copy(denh.at[pl.ds(nbase, NN), :], denv)
        pltpu.sync_copy(bias, bv)
        zz = jnp.zeros((16,), jnp.float32)

        def nbody(i, _):
            n = nbase + i

            @pl.when(n < N)
            def _():
                sv = startv[pl.ds(i, 16)]
                s = sv[0]
                e1 = sv[1]
                b0 = s // EB
                b1 = (e1 + EB - 1) // EB

                def zb(j, _):
                    acc[pl.ds(j * 16, 16)] = zz
                    return 0

                lax.fori_loop(0, HC // 16, zb, 0)
                am = [amaxv[i, pl.ds(k * 16, 16)] for k in range(3)]
                iv = [1.0 / (denv[i, pl.ds(k * 16, 16)] + 1e-16)
                      for k in range(3)]

                def bb(bi, _):
                    off = pl.multiple_of(bi * EB, EB)
                    pltpu.sync_copy(ssrcp.at[pl.ds(off, EB)], idxb)
                    pltpu.async_copy(xl.at[idxb], glb, sem).wait()
                    pltpu.sync_copy(alpha.at[pl.ds(off, EB), :], ab)
                    for e in range(EB):
                        valid = ((off + e) >= s) & ((off + e) < e1)
                        avecs = []
                        for k in range(3):
                            r = ab[e, pl.ds(k * 16, 16)]
                            a = jnp.exp(r - am[k]) * iv[k]
                            avecs.append(jnp.where(valid, a, 0.0))

                        def cc(j, _, avecs=avecs):
                            for h in range(H):
                                asc = avecs[h // 16][h % 16]
                                seg = glb[e, pl.ds(h * C + j * 16, 16)]
                                plsc.addupdate(
                                    acc.at[pl.ds(h * C + j * 16, 16)],
                                    asc * seg)
                            return 0

                        lax.fori_loop(0, C // 16, cc, 0)
                    return 0

                lax.fori_loop(b0, b1, bb, 0)
                pltpu.sync_copy(xres.at[n], xrow)

                def eb(j, _):
                    v = (acc[pl.ds(j * 16, 16)] + xrow[pl.ds(j * 16, 16)]
                         + bv[pl.ds(j * 16, 16)])
                    sv = jnp.where(v > 0.0, SELU_L * v,
                                   SELU_L * SELU_A * (jnp.exp(v) - 1.0))
                    orow[pl.ds(j * 16, 16)] = sv
                    return 0

                lax.fori_loop(0, HC // 16, eb, 0)
                pltpu.sync_copy(orow, out.at[n])

            return 0

        lax.fori_loop(0, NN, nbody, 0)

    return k5


_K3 = {c: _make_k3(c) for c in (16, 32, 64)}
_K4 = _make_k4()
_K5 = {c: _make_k5(c) for c in (16, 32, 64)}


def _gat_layer(p, x, edge_attr, ssrc, sdst, perm, ssrc_p, start, C):
    xl = _matmul(x, p['Wl'])
    xr = _matmul(x, p['Wr'])
    xres = _matmul(x, p['Wres'])
    ew = _matmul(edge_attr, p['We'])
    attp = jnp.concatenate(
        [p['att'], jnp.zeros((HP - H, C), jnp.float32)], axis=0).reshape(-1)
    alpha = _K3[C](xl, xr, ew, ssrc, sdst, perm, attp)
    amax, den = _K4(alpha, start)
    return _K5[C](alpha, start, amax, den, ssrc_p, xl, xres, p['b'])


def _bn(h, p):
    mu = jnp.mean(h, axis=0)
    var = jnp.var(h, axis=0)
    return p['g'] * (h - mu) / jnp.sqrt(var + 1e-5) + p['b']


def kernel(x, edge_index, edge_attr, batch, params):
    src, dst = edge_index[0], edge_index[1]
    eidx = jnp.arange(E, dtype=jnp.int32)
    sdst, perm = lax.sort_key_val(dst, eidx)
    ssrc = jnp.take(src, perm)
    start = jnp.searchsorted(
        sdst, jnp.arange(NP + 16, dtype=jnp.int32), side='left'
    ).astype(jnp.int32)
    ssrc_p = jnp.concatenate([ssrc, jnp.zeros((EB,), jnp.int32)])

    h = _gat_layer(params['gat1'], x, edge_attr, ssrc, sdst, perm,
                   ssrc_p, start, 16)
    h = _gat_layer(params['gat2'], h, edge_attr, ssrc, sdst, perm,
                   ssrc_p, start, 32)
    h = _gat_layer(params['gat3'], h, edge_attr, ssrc, sdst, perm,
                   ssrc_p, start, 64)
    h = _bn(_matmul(h, params['lin1']['W']) + params['lin1']['b'],
            params['bn1'])
    h = jax.nn.selu(h)
    h = _bn(_matmul(h, params['lin2']['W']) + params['lin2']['b'],
            params['bn2'])
    h = jax.nn.selu(h)
    RF = _matmul(h, params['lin3']['W']) + params['lin3']['b']
    BB = _matmul(h, params['lin4']['W']) + params['lin4']['b']
    P = _matmul(h, params['lin5']['W']) + params['lin5']['b']
    return (RF, BB, P)
